# baseline (device time: 23905 ns/iter reference)
import jax
import jax.numpy as jnp
from jax import lax
from jax.experimental import pallas as pl
from jax.experimental.pallas import tpu as pltpu

N_CHUNKS = 8
RELAY_LAG = 2
Q = 64
U = 8
A_ROWS = Q - U
H_ROWS = Q // 2
SEND_ROWS = Q + 2 * U


def kernel(x, dy):
    k, m = x.shape
    _, f = dy.shape
    m_half = m // 2
    fc = f // N_CHUNKS

    def body(x_ref, dy_ref, out_ref, psend, pown, xrecv,
             xqs, xqr, xus, xur, xvs, xvr,
             yas, yar, ybs, ybr, zas, zar, zbs, zbr):
        my_x = lax.axis_index("x")
        my_y = lax.axis_index("y")
        my_z = lax.axis_index("z")
        zlo = lax.rem(my_z, 2)
        pair_z = my_z + 1 - 2 * zlo

        x_partner = (1 - my_x, my_y, my_z)
        y_nb = (my_x, 1 - my_y, my_z)
        z_nb = (my_x, my_y, pair_z)

        q = 2 * zlo + my_y
        qy = 2 * zlo + (1 - my_y)
        qz = 2 * (1 - zlo) + my_y

        r_q = pl.multiple_of(Q * q, Q)
        r_qy_tail = pl.multiple_of(Q * qy + A_ROWS, 8)
        r_qz_tail = pl.multiple_of(Q * qz + A_ROWS, 8)
        r_qy_half = pl.multiple_of(Q * qy + H_ROWS, H_ROWS)
        r_qz = pl.multiple_of(Q * qz, Q)

        barrier_sem = pltpu.get_barrier_semaphore()
        for nbr in (x_partner, y_nb, z_nb):
            pl.semaphore_signal(
                barrier_sem, inc=1,
                device_id=nbr, device_id_type=pl.DeviceIdType.MESH,
            )
        pl.semaphore_wait(barrier_sem, 3)

        x_rdmas = []
        for c in range(N_CHUNKS):
            cols = pl.ds(c * fc, fc)
            psend[:, cols] = lax.dot_general(
                x_ref[:, pl.ds((1 - my_x) * m_half, m_half)], dy_ref[:, cols],
                (((0,), (0,)), ((), ())),
                preferred_element_type=jnp.float32,
            )
            trip = []
            for (src_r, n_rows, dst_r, ss, rr) in (
                (r_q, Q, 0, xqs, xqr),
                (r_qy_tail, U, Q, xus, xur),
                (r_qz_tail, U, Q + U, xvs, xvr),
            ):
                rdma = pltpu.make_async_remote_copy(
                    src_ref=psend.at[pl.ds(src_r, n_rows), cols],
                    dst_ref=xrecv.at[pl.ds(dst_r, n_rows), cols],
                    send_sem=ss.at[c], recv_sem=rr.at[c],
                    device_id=x_partner, device_id_type=pl.DeviceIdType.MESH,
                )
                rdma.start()
                trip.append(rdma)
            x_rdmas.append(trip)

        pown[...] = lax.dot_general(
            x_ref[:, pl.ds(my_x * m_half, m_half)], dy_ref[...],
            (((0,), (0,)), ((), ())),
            preferred_element_type=jnp.float32,
        )

        ya_rdmas, yb_rdmas, za_rdmas, zb_rdmas = [], [], [], []

        def start_relays(c):
            cols = pl.ds(c * fc, fc)
            ya_rdmas[c].wait_recv()
            rdma = pltpu.make_async_remote_copy(
                src_ref=out_ref.at[pl.ds(r_qy_half, H_ROWS), cols],
                dst_ref=out_ref.at[pl.ds(r_qy_half, H_ROWS), cols],
                send_sem=zbs.at[c], recv_sem=zbr.at[c],
                device_id=z_nb, device_id_type=pl.DeviceIdType.MESH,
            )
            rdma.start()
            zb_rdmas.append(rdma)
            za_rdmas[c].wait_recv()
            rdma = pltpu.make_async_remote_copy(
                src_ref=out_ref.at[pl.ds(r_qz, H_ROWS), cols],
                dst_ref=out_ref.at[pl.ds(r_qz, H_ROWS), cols],
                send_sem=ybs.at[c], recv_sem=ybr.at[c],
                device_id=y_nb, device_id_type=pl.DeviceIdType.MESH,
            )
            rdma.start()
            yb_rdmas.append(rdma)

        for c in range(N_CHUNKS):
            cols = pl.ds(c * fc, fc)
            for rdma in x_rdmas[c]:
                rdma.wait_recv()
            out_ref[pl.ds(r_q, Q), cols] = (
                pown[pl.ds(r_q, Q), cols] + xrecv[0:Q, cols]
            )
            out_ref[pl.ds(r_qy_tail, U), cols] = (
                pown[pl.ds(r_qy_tail, U), cols] + xrecv[Q:Q + U, cols]
            )
            out_ref[pl.ds(r_qz_tail, U), cols] = (
                pown[pl.ds(r_qz_tail, U), cols] + xrecv[Q + U:SEND_ROWS, cols]
            )
            rdma = pltpu.make_async_remote_copy(
                src_ref=out_ref.at[pl.ds(r_q, A_ROWS), cols],
                dst_ref=out_ref.at[pl.ds(r_q, A_ROWS), cols],
                send_sem=yas.at[c], recv_sem=yar.at[c],
                device_id=y_nb, device_id_type=pl.DeviceIdType.MESH,
            )
            rdma.start()
            ya_rdmas.append(rdma)
            rdma = pltpu.make_async_remote_copy(
                src_ref=out_ref.at[pl.ds(r_q, A_ROWS), cols],
                dst_ref=out_ref.at[pl.ds(r_q, A_ROWS), cols],
                send_sem=zas.at[c], recv_sem=zar.at[c],
                device_id=z_nb, device_id_type=pl.DeviceIdType.MESH,
            )
            rdma.start()
            za_rdmas.append(rdma)
            if c >= RELAY_LAG:
                start_relays(c - RELAY_LAG)

        for c in range(N_CHUNKS - RELAY_LAG, N_CHUNKS):
            start_relays(c)

        for c in range(N_CHUNKS):
            yb_rdmas[c].wait_recv()
            zb_rdmas[c].wait_recv()
        for c in range(N_CHUNKS):
            for rdma in x_rdmas[c]:
                rdma.wait_send()
            ya_rdmas[c].wait_send()
            yb_rdmas[c].wait_send()
            za_rdmas[c].wait_send()
            zb_rdmas[c].wait_send()

    sem = pltpu.SemaphoreType.DMA((N_CHUNKS,))
    return pl.pallas_call(
        body,
        out_shape=jax.ShapeDtypeStruct((m_half, f), jnp.float32),
        in_specs=[
            pl.BlockSpec(memory_space=pltpu.VMEM),
            pl.BlockSpec(memory_space=pltpu.VMEM),
        ],
        out_specs=pl.BlockSpec(memory_space=pltpu.VMEM),
        scratch_shapes=[
            pltpu.VMEM((m_half, f), jnp.float32),
            pltpu.VMEM((m_half, f), jnp.float32),
            pltpu.VMEM((SEND_ROWS, f), jnp.float32),
            sem, sem,
            sem, sem,
            sem, sem,
            sem, sem,
            sem, sem,
            sem, sem,
            sem, sem,
        ],
        compiler_params=pltpu.CompilerParams(collective_id=0),
    )(x, dy)


# device time: 23437 ns/iter; 1.0200x vs baseline; 1.0200x over previous
import jax
import jax.numpy as jnp
from jax import lax
from jax.experimental import pallas as pl
from jax.experimental.pallas import tpu as pltpu

N_CHUNKS = 8
RELAY_LAG = 2
Q = 64
U = 16
A_ROWS = Q - U
H_ROWS = Q // 2
SEND_ROWS = Q + 2 * U


def kernel(x, dy):
    k, m = x.shape
    _, f = dy.shape
    m_half = m // 2
    fc = f // N_CHUNKS

    def body(x_ref, dy_ref, out_ref, psend, pown, xrecv,
             xqs, xqr, xus, xur, xvs, xvr,
             yas, yar, ybs, ybr, zas, zar, zbs, zbr):
        my_x = lax.axis_index("x")
        my_y = lax.axis_index("y")
        my_z = lax.axis_index("z")
        zlo = lax.rem(my_z, 2)
        pair_z = my_z + 1 - 2 * zlo

        x_partner = (1 - my_x, my_y, my_z)
        y_nb = (my_x, 1 - my_y, my_z)
        z_nb = (my_x, my_y, pair_z)

        q = 2 * zlo + my_y
        qy = 2 * zlo + (1 - my_y)
        qz = 2 * (1 - zlo) + my_y

        r_q = pl.multiple_of(Q * q, Q)
        r_qy_tail = pl.multiple_of(Q * qy + A_ROWS, 8)
        r_qz_tail = pl.multiple_of(Q * qz + A_ROWS, 8)
        r_qy_half = pl.multiple_of(Q * qy + H_ROWS, H_ROWS)
        r_qz = pl.multiple_of(Q * qz, Q)

        psend[:, pl.ds(0, fc)] = lax.dot_general(
            x_ref[:, pl.ds((1 - my_x) * m_half, m_half)], dy_ref[:, pl.ds(0, fc)],
            (((0,), (0,)), ((), ())),
            preferred_element_type=jnp.float32,
        )

        barrier_sem = pltpu.get_barrier_semaphore()
        for nbr in (x_partner, y_nb, z_nb):
            pl.semaphore_signal(
                barrier_sem, inc=1,
                device_id=nbr, device_id_type=pl.DeviceIdType.MESH,
            )
        pl.semaphore_wait(barrier_sem, 3)

        x_rdmas = []
        for c in range(N_CHUNKS):
            cols = pl.ds(c * fc, fc)
            if c > 0:
                psend[:, cols] = lax.dot_general(
                    x_ref[:, pl.ds((1 - my_x) * m_half, m_half)], dy_ref[:, cols],
                    (((0,), (0,)), ((), ())),
                    preferred_element_type=jnp.float32,
                )
            trip = []
            for (src_r, n_rows, dst_r, ss, rr) in (
                (r_q, Q, 0, xqs, xqr),
                (r_qy_tail, U, Q, xus, xur),
                (r_qz_tail, U, Q + U, xvs, xvr),
            ):
                rdma = pltpu.make_async_remote_copy(
                    src_ref=psend.at[pl.ds(src_r, n_rows), cols],
                    dst_ref=xrecv.at[pl.ds(dst_r, n_rows), cols],
                    send_sem=ss.at[c], recv_sem=rr.at[c],
                    device_id=x_partner, device_id_type=pl.DeviceIdType.MESH,
                )
                rdma.start()
                trip.append(rdma)
            x_rdmas.append(trip)

        pown[...] = lax.dot_general(
            x_ref[:, pl.ds(my_x * m_half, m_half)], dy_ref[...],
            (((0,), (0,)), ((), ())),
            preferred_element_type=jnp.float32,
        )

        ya_rdmas, yb_rdmas, za_rdmas, zb_rdmas = [], [], [], []

        def start_relays(c):
            cols = pl.ds(c * fc, fc)
            ya_rdmas[c].wait_recv()
            rdma = pltpu.make_async_remote_copy(
                src_ref=out_ref.at[pl.ds(r_qy_half, H_ROWS), cols],
                dst_ref=out_ref.at[pl.ds(r_qy_half, H_ROWS), cols],
                send_sem=zbs.at[c], recv_sem=zbr.at[c],
                device_id=z_nb, device_id_type=pl.DeviceIdType.MESH,
            )
            rdma.start()
            zb_rdmas.append(rdma)
            za_rdmas[c].wait_recv()
            rdma = pltpu.make_async_remote_copy(
                src_ref=out_ref.at[pl.ds(r_qz, H_ROWS), cols],
                dst_ref=out_ref.at[pl.ds(r_qz, H_ROWS), cols],
                send_sem=ybs.at[c], recv_sem=ybr.at[c],
                device_id=y_nb, device_id_type=pl.DeviceIdType.MESH,
            )
            rdma.start()
            yb_rdmas.append(rdma)

        for c in range(N_CHUNKS):
            cols = pl.ds(c * fc, fc)
            for rdma in x_rdmas[c]:
                rdma.wait_recv()
            out_ref[pl.ds(r_q, Q), cols] = (
                pown[pl.ds(r_q, Q), cols] + xrecv[0:Q, cols]
            )
            out_ref[pl.ds(r_qy_tail, U), cols] = (
                pown[pl.ds(r_qy_tail, U), cols] + xrecv[Q:Q + U, cols]
            )
            out_ref[pl.ds(r_qz_tail, U), cols] = (
                pown[pl.ds(r_qz_tail, U), cols] + xrecv[Q + U:SEND_ROWS, cols]
            )
            rdma = pltpu.make_async_remote_copy(
                src_ref=out_ref.at[pl.ds(r_q, A_ROWS), cols],
                dst_ref=out_ref.at[pl.ds(r_q, A_ROWS), cols],
                send_sem=yas.at[c], recv_sem=yar.at[c],
                device_id=y_nb, device_id_type=pl.DeviceIdType.MESH,
            )
            rdma.start()
            ya_rdmas.append(rdma)
            rdma = pltpu.make_async_remote_copy(
                src_ref=out_ref.at[pl.ds(r_q, A_ROWS), cols],
                dst_ref=out_ref.at[pl.ds(r_q, A_ROWS), cols],
                send_sem=zas.at[c], recv_sem=zar.at[c],
                device_id=z_nb, device_id_type=pl.DeviceIdType.MESH,
            )
            rdma.start()
            za_rdmas.append(rdma)
            if c >= RELAY_LAG:
                start_relays(c - RELAY_LAG)

        for c in range(N_CHUNKS - RELAY_LAG, N_CHUNKS):
            start_relays(c)

        for c in range(N_CHUNKS):
            yb_rdmas[c].wait_recv()
            zb_rdmas[c].wait_recv()
        for c in range(N_CHUNKS):
            for rdma in x_rdmas[c]:
                rdma.wait_send()
            ya_rdmas[c].wait_send()
            yb_rdmas[c].wait_send()
            za_rdmas[c].wait_send()
            zb_rdmas[c].wait_send()

    sem = pltpu.SemaphoreType.DMA((N_CHUNKS,))
    return pl.pallas_call(
        body,
        out_shape=jax.ShapeDtypeStruct((m_half, f), jnp.float32),
        in_specs=[
            pl.BlockSpec(memory_space=pltpu.VMEM),
            pl.BlockSpec(memory_space=pltpu.VMEM),
        ],
        out_specs=pl.BlockSpec(memory_space=pltpu.VMEM),
        scratch_shapes=[
            pltpu.VMEM((m_half, f), jnp.float32),
            pltpu.VMEM((m_half, f), jnp.float32),
            pltpu.VMEM((SEND_ROWS, f), jnp.float32),
            sem, sem,
            sem, sem,
            sem, sem,
            sem, sem,
            sem, sem,
            sem, sem,
            sem, sem,
        ],
        compiler_params=pltpu.CompilerParams(collective_id=0),
    )(x, dy)
